# baseline (speedup 1.0000x reference)
import jax
import jax.numpy as jnp
from jax.experimental import pallas as pl
from jax.experimental.pallas import tpu as pltpu

E = 8
H = 1024
I = 1024
T = 32
R = 128
CH = R * H
N1 = I // R


def _bf(a):
    return a.astype(jnp.bfloat16)


def _stack(v):
    return jnp.concatenate([v[:, r * H : (r + 1) * H] for r in range(R)], axis=0)


def _moe_kernel(x_ref, rw_ref, ws_ref, out_ref, h1t_ref, y_ref, fwb_ref, st_ref):
    j = pl.program_id(0)

    @pl.when(j == 0)
    def _router():
        logits = jax.lax.dot_general(
            x_ref[...], rw_ref[...], (((1,), (1,)), ((), ())),
            preferred_element_type=jnp.float32,
            precision=jax.lax.Precision.HIGHEST,
        )
        m = jnp.max(logits, axis=-1, keepdims=True)
        p = jnp.exp(logits - m)
        s = p / jnp.sum(p, axis=-1, keepdims=True)
        lane = jax.lax.broadcasted_iota(jnp.int32, (T, E), 1)
        m1 = jnp.max(s, axis=-1, keepdims=True)
        i1 = jnp.min(jnp.where(s == m1, lane, E), axis=-1, keepdims=True)
        s_rest = jnp.where(lane == i1, -1.0, s)
        m2 = jnp.max(s_rest, axis=-1, keepdims=True)
        i2 = jnp.min(jnp.where(s_rest == m2, lane, E), axis=-1, keepdims=True)
        fw = jnp.where(lane == i1, m1, 0.0) + jnp.where(lane == i2, m2, 0.0)
        fw = fw / (m1 + m2)
        fwt = fw.T
        fwb_ref[...] = jnp.concatenate([fwt for _ in range(R)], axis=0)

    @pl.when(j < N1)
    def _w1():
        w_stack = _stack(ws_ref[...])
        h1t = jax.lax.dot_general(
            _bf(w_stack), _bf(x_ref[...]), (((1,), (1,)), ((), ())),
            preferred_element_type=jnp.float32,
        )
        h1t_ref[pl.ds(j * 8 * R, 8 * R), :] = h1t

    @pl.when(jnp.logical_and(j >= N1, j < 2 * N1))
    def _v1():
        jj = j - N1
        w_stack = _stack(ws_ref[...])
        h2t = jax.lax.dot_general(
            _bf(w_stack), _bf(x_ref[...]), (((1,), (1,)), ((), ())),
            preferred_element_type=jnp.float32,
        )
        h1t = h1t_ref[pl.ds(jj * 8 * R, 8 * R), :]
        at = (h1t * jax.lax.logistic(h1t)) * h2t * fwb_ref[...]
        st_ref[:, 0:T] = at
        y_ref[pl.ds(jj * R, R), :] = jnp.concatenate(
            [st_ref[pl.Slice(e, R, E), :][:, 0:T] for e in range(E)], axis=1
        )

    @pl.when(j >= 2 * N1)
    def _w2():
        jj = j - 2 * N1
        w_stack = _stack(ws_ref[...])
        of = jax.lax.dot_general(
            _bf(w_stack), _bf(y_ref[...]), (((1,), (0,)), ((), ())),
            preferred_element_type=jnp.float32,
        )
        st_ref[...] = of[:, 0:128]
        och = st_ref[pl.Slice(0, R, E), :][:, 0:T]
        for e in range(1, 4):
            och = och + st_ref[pl.Slice(e, R, E), :][:, e * T : (e + 1) * T]
        st_ref[...] = of[:, 128:256]
        for e in range(4, E):
            och = och + st_ref[pl.Slice(e, R, E), :][:, (e - 4) * T : (e - 3) * T]
        out_ref[:, pl.ds(jj * R, R)] = och.T


def kernel(index, hidden_states, router_w, ws):
    del index
    x = hidden_states.reshape(T, H)
    rw = router_w.reshape(E, H)
    ws2d = ws.reshape(E, 3 * I * H)
    return pl.pallas_call(
        _moe_kernel,
        grid=(3 * N1,),
        in_specs=[
            pl.BlockSpec((T, H), lambda j: (0, 0)),
            pl.BlockSpec((E, H), lambda j: (0, 0)),
            pl.BlockSpec((E, CH), lambda j: (0, j)),
        ],
        out_specs=pl.BlockSpec((T, H), lambda j: (0, 0)),
        out_shape=jax.ShapeDtypeStruct((T, H), jnp.float32),
        scratch_shapes=[
            pltpu.VMEM((E * I, T), jnp.float32),
            pltpu.VMEM((I, E * T), jnp.float32),
            pltpu.VMEM((E * R, T), jnp.float32),
            pltpu.VMEM((E * R, 128), jnp.float32),
        ],
    )(x, rw, ws2d)


# speedup vs baseline: 1.1646x; 1.1646x over previous
import jax
import jax.numpy as jnp
from jax.experimental import pallas as pl
from jax.experimental.pallas import tpu as pltpu

E = 8
H = 1024
I = 1024
T = 32
R = 512
CH = R * H
N1 = I // R


def _bf(a):
    return a.astype(jnp.bfloat16)


def _stack(v):
    return jnp.concatenate([v[:, r * H : (r + 1) * H] for r in range(R)], axis=0)


def _moe_kernel(x_ref, rw_ref, ws_ref, out_ref, h1t_ref, y_ref, fwb_ref, st_ref):
    j = pl.program_id(0)

    @pl.when(j == 0)
    def _router():
        logits = jax.lax.dot_general(
            x_ref[...], rw_ref[...], (((1,), (1,)), ((), ())),
            preferred_element_type=jnp.float32,
            precision=jax.lax.Precision.HIGHEST,
        )
        m = jnp.max(logits, axis=-1, keepdims=True)
        p = jnp.exp(logits - m)
        s = p / jnp.sum(p, axis=-1, keepdims=True)
        lane = jax.lax.broadcasted_iota(jnp.int32, (T, E), 1)
        m1 = jnp.max(s, axis=-1, keepdims=True)
        i1 = jnp.min(jnp.where(s == m1, lane, E), axis=-1, keepdims=True)
        s_rest = jnp.where(lane == i1, -1.0, s)
        m2 = jnp.max(s_rest, axis=-1, keepdims=True)
        i2 = jnp.min(jnp.where(s_rest == m2, lane, E), axis=-1, keepdims=True)
        fw = jnp.where(lane == i1, m1, 0.0) + jnp.where(lane == i2, m2, 0.0)
        fw = fw / (m1 + m2)
        fwt = fw.T
        fwb_ref[...] = jnp.concatenate([fwt for _ in range(R)], axis=0)

    @pl.when(j < N1)
    def _w1():
        w_stack = _stack(ws_ref[...])
        h1t = jax.lax.dot_general(
            _bf(w_stack), _bf(x_ref[...]), (((1,), (1,)), ((), ())),
            preferred_element_type=jnp.float32,
        )
        h1t_ref[pl.ds(j * 8 * R, 8 * R), :] = h1t

    @pl.when(jnp.logical_and(j >= N1, j < 2 * N1))
    def _v1():
        jj = j - N1
        w_stack = _stack(ws_ref[...])
        h2t = jax.lax.dot_general(
            _bf(w_stack), _bf(x_ref[...]), (((1,), (1,)), ((), ())),
            preferred_element_type=jnp.float32,
        )
        h1t = h1t_ref[pl.ds(jj * 8 * R, 8 * R), :]
        at = (h1t * jax.lax.logistic(h1t)) * h2t * fwb_ref[...]
        st_ref[:, 0:T] = at
        y_ref[pl.ds(jj * R, R), :] = jnp.concatenate(
            [st_ref[pl.Slice(e, R, E), :][:, 0:T] for e in range(E)], axis=1
        )

    @pl.when(j >= 2 * N1)
    def _w2():
        jj = j - 2 * N1
        w_stack = _stack(ws_ref[...])
        of = jax.lax.dot_general(
            _bf(w_stack), _bf(y_ref[...]), (((1,), (0,)), ((), ())),
            preferred_element_type=jnp.float32,
        )
        st_ref[...] = of[:, 0:128]
        och = st_ref[pl.Slice(0, R, E), :][:, 0:T]
        for e in range(1, 4):
            och = och + st_ref[pl.Slice(e, R, E), :][:, e * T : (e + 1) * T]
        st_ref[...] = of[:, 128:256]
        for e in range(4, E):
            och = och + st_ref[pl.Slice(e, R, E), :][:, (e - 4) * T : (e - 3) * T]
        out_ref[:, pl.ds(jj * R, R)] = och.T


def kernel(index, hidden_states, router_w, ws):
    del index
    x = hidden_states.reshape(T, H)
    rw = router_w.reshape(E, H)
    ws2d = ws.reshape(E, 3 * I * H)
    return pl.pallas_call(
        _moe_kernel,
        grid=(3 * N1,),
        in_specs=[
            pl.BlockSpec((T, H), lambda j: (0, 0)),
            pl.BlockSpec((E, H), lambda j: (0, 0)),
            pl.BlockSpec((E, CH), lambda j: (0, j)),
        ],
        out_specs=pl.BlockSpec((T, H), lambda j: (0, 0)),
        out_shape=jax.ShapeDtypeStruct((T, H), jnp.float32),
        scratch_shapes=[
            pltpu.VMEM((E * I, T), jnp.float32),
            pltpu.VMEM((I, E * T), jnp.float32),
            pltpu.VMEM((E * R, T), jnp.float32),
            pltpu.VMEM((E * R, 128), jnp.float32),
        ],
    )(x, rw, ws2d)


# speedup vs baseline: 1.2006x; 1.0309x over previous
import jax
import jax.numpy as jnp
from jax.experimental import pallas as pl
from jax.experimental.pallas import tpu as pltpu

E = 8
H = 1024
I = 1024
T = 32
R = 256
CH = R * H
N1 = I // R


def _bf(a):
    return a.astype(jnp.bfloat16)


def _stack(v):
    return jnp.concatenate([v[:, r * H : (r + 1) * H] for r in range(R)], axis=0)


def _moe_kernel(x_ref, rw_ref, ws_ref, out_ref, h1t_ref, y_ref, fwb_ref, st_ref):
    j = pl.program_id(0)

    @pl.when(j == 0)
    def _router():
        logits = jax.lax.dot_general(
            x_ref[...], rw_ref[...], (((1,), (1,)), ((), ())),
            preferred_element_type=jnp.float32,
            precision=jax.lax.Precision.HIGHEST,
        )
        m = jnp.max(logits, axis=-1, keepdims=True)
        p = jnp.exp(logits - m)
        s = p / jnp.sum(p, axis=-1, keepdims=True)
        lane = jax.lax.broadcasted_iota(jnp.int32, (T, E), 1)
        m1 = jnp.max(s, axis=-1, keepdims=True)
        i1 = jnp.min(jnp.where(s == m1, lane, E), axis=-1, keepdims=True)
        s_rest = jnp.where(lane == i1, -1.0, s)
        m2 = jnp.max(s_rest, axis=-1, keepdims=True)
        i2 = jnp.min(jnp.where(s_rest == m2, lane, E), axis=-1, keepdims=True)
        fw = jnp.where(lane == i1, m1, 0.0) + jnp.where(lane == i2, m2, 0.0)
        fw = fw / (m1 + m2)
        fwt = fw.T
        fwb_ref[...] = jnp.concatenate([fwt for _ in range(R)], axis=0)

    @pl.when(j < N1)
    def _w1():
        w_stack = _stack(ws_ref[...])
        h1t = jax.lax.dot_general(
            _bf(w_stack), _bf(x_ref[...]), (((1,), (1,)), ((), ())),
            preferred_element_type=jnp.float32,
        )
        h1t_ref[pl.ds(j * 8 * R, 8 * R), :] = h1t

    @pl.when(jnp.logical_and(j >= N1, j < 2 * N1))
    def _v1():
        jj = j - N1
        w_stack = _stack(ws_ref[...])
        h2t = jax.lax.dot_general(
            _bf(w_stack), _bf(x_ref[...]), (((1,), (1,)), ((), ())),
            preferred_element_type=jnp.float32,
        )
        h1t = h1t_ref[pl.ds(jj * 8 * R, 8 * R), :]
        at = (h1t * jax.lax.logistic(h1t)) * h2t * fwb_ref[...]
        st_ref[:, 0:T] = at
        y_ref[pl.ds(jj * R, R), :] = jnp.concatenate(
            [st_ref[pl.Slice(e, R, E), :][:, 0:T] for e in range(E)], axis=1
        )

    @pl.when(j >= 2 * N1)
    def _w2():
        jj = j - 2 * N1
        w_stack = _stack(ws_ref[...])
        of = jax.lax.dot_general(
            _bf(w_stack), _bf(y_ref[...]), (((1,), (0,)), ((), ())),
            preferred_element_type=jnp.float32,
        )
        st_ref[...] = of[:, 0:128]
        och = st_ref[pl.Slice(0, R, E), :][:, 0:T]
        for e in range(1, 4):
            och = och + st_ref[pl.Slice(e, R, E), :][:, e * T : (e + 1) * T]
        st_ref[...] = of[:, 128:256]
        for e in range(4, E):
            och = och + st_ref[pl.Slice(e, R, E), :][:, (e - 4) * T : (e - 3) * T]
        out_ref[:, pl.ds(jj * R, R)] = och.T


def kernel(index, hidden_states, router_w, ws):
    del index
    x = hidden_states.reshape(T, H)
    rw = router_w.reshape(E, H)
    ws2d = ws.reshape(E, 3 * I * H)
    return pl.pallas_call(
        _moe_kernel,
        grid=(3 * N1,),
        in_specs=[
            pl.BlockSpec((T, H), lambda j: (0, 0)),
            pl.BlockSpec((E, H), lambda j: (0, 0)),
            pl.BlockSpec((E, CH), lambda j: (0, j)),
        ],
        out_specs=pl.BlockSpec((T, H), lambda j: (0, 0)),
        out_shape=jax.ShapeDtypeStruct((T, H), jnp.float32),
        scratch_shapes=[
            pltpu.VMEM((E * I, T), jnp.float32),
            pltpu.VMEM((I, E * T), jnp.float32),
            pltpu.VMEM((E * R, T), jnp.float32),
            pltpu.VMEM((E * R, 128), jnp.float32),
        ],
    )(x, rw, ws2d)
